# Initial kernel scaffold; baseline (speedup 1.0000x reference)
import jax
import jax.numpy as jnp
from jax.experimental import pallas as pl


def kernel(stacked_coords, stacked_lattices, stacked_nbr_lists):
    raise NotImplementedError("write your pallas kernel here")


# speedup vs baseline: 21.0634x; 21.0634x over previous
import functools

import jax
import jax.numpy as jnp
import numpy as np
from jax import lax
from jax.experimental import pallas as pl
from jax.experimental.pallas import tpu as pltpu
from jax.experimental.pallas import tpu_sc as plsc

DMIN, DMAX, STEP = 0.0, 8.0, 0.25
VAR = STEP
NFILT = 33
NC, NS = 2, 16
SUBS_PER_CASE = 8


def _sc_dist2(coords_t, nbr_t, lat_pad, B, N, M):
    NM = N * M
    EPW = NM // SUBS_PER_CASE
    APW = N // SUBS_PER_CASE
    mesh = plsc.VectorSubcoreMesh(
        core_axis_name="c", subcore_axis_name="s", num_cores=NC, num_subcores=NS
    )

    @functools.partial(
        pl.kernel,
        mesh=mesh,
        compiler_params=pltpu.CompilerParams(needs_layout_passes=False),
        out_type=(
            jax.ShapeDtypeStruct((B * NM,), jnp.float32),
            jax.ShapeDtypeStruct((B * NM,), jnp.float32),
        ),
        scratch_types=[
            pltpu.VMEM((N,), jnp.float32),
            pltpu.VMEM((N,), jnp.float32),
            pltpu.VMEM((N,), jnp.float32),
            pltpu.VMEM((EPW,), jnp.int32),
            pltpu.VMEM((EPW,), jnp.float32),
            pltpu.VMEM((48,), jnp.float32),
        ],
    )
    def k(coords_hbm, nbr_hbm, lat_hbm, out1, out2, xv, yv, zv, idxv, ov, latv):
        c = lax.axis_index("c")
        s = lax.axis_index("s")
        b = s // SUBS_PER_CASE
        sub = s % SUBS_PER_CASE
        base = sub * EPW
        atoms0 = sub * APW
        case = c * B + b

        pltpu.sync_copy(coords_hbm.at[pl.ds((case * 3 + 0) * N, N)], xv)
        pltpu.sync_copy(coords_hbm.at[pl.ds((case * 3 + 1) * N, N)], yv)
        pltpu.sync_copy(coords_hbm.at[pl.ds((case * 3 + 2) * N, N)], zv)
        pltpu.sync_copy(nbr_hbm.at[pl.ds(case * NM + base, EPW)], idxv)
        pltpu.sync_copy(lat_hbm.at[pl.ds(case * 48, 48)], latv)

        lxv = latv[pl.ds(0, 16)]
        lyv = latv[pl.ds(16, 16)]
        lzv = latv[pl.ds(32, 16)]
        hxv = lxv * 0.5
        hyv = lyv * 0.5
        hzv = lzv * 0.5

        def body(a, _):
            gidx = jnp.full((16,), atoms0 + a, jnp.int32)
            ax = plsc.load_gather(xv, [gidx])
            ay = plsc.load_gather(yv, [gidx])
            az = plsc.load_gather(zv, [gidx])
            for h in range(M // 16):
                off = a * M + h * 16
                idx = idxv[pl.ds(off, 16)]
                dx = jnp.abs(ax - plsc.load_gather(xv, [idx]))
                dy = jnp.abs(ay - plsc.load_gather(yv, [idx]))
                dz = jnp.abs(az - plsc.load_gather(zv, [idx]))
                dx = jnp.where(dx > hxv, dx - lxv, dx)
                dy = jnp.where(dy > hyv, dy - lyv, dy)
                dz = jnp.where(dz > hzv, dz - lzv, dz)
                ov[pl.ds(off, 16)] = dx * dx + dy * dy + dz * dz
            return 0

        lax.fori_loop(0, APW, body, 0, unroll=2)

        @pl.when(c == 0)
        def _():
            pltpu.sync_copy(ov, out1.at[pl.ds(b * NM + base, EPW)])

        @pl.when(c == 1)
        def _():
            pltpu.sync_copy(ov, out2.at[pl.ds(b * NM + base, EPW)])

    return k(coords_t, nbr_t, lat_pad)


def _tc_expand(d2_rows, sel):
    R, M = d2_rows.shape
    W = M * NFILT
    RB = 200 if R % 200 == 0 else 8
    assert R % RB == 0
    inv_var2 = 1.0 / (VAR * VAR)

    def body(d2_ref, sel_ref, o_ref):
        d = jnp.sqrt(d2_ref[...])
        drep = lax.dot_general(d, sel_ref[...],
                               (((1,), (0,)), ((), ())),
                               precision=lax.Precision.HIGHEST,
                               preferred_element_type=jnp.float32)
        col = lax.broadcasted_iota(jnp.int32, (RB, W), 1)
        f = (col % NFILT).astype(jnp.float32) * STEP
        diff = drep - f
        o_ref[...] = jnp.exp(diff * diff * (-inv_var2))

    return pl.pallas_call(
        body,
        grid=(R // RB,),
        in_specs=[
            pl.BlockSpec((RB, M), lambda i: (i, 0)),
            pl.BlockSpec((M, W), lambda i: (0, 0)),
        ],
        out_specs=pl.BlockSpec((RB, W), lambda i: (i, 0)),
        out_shape=jax.ShapeDtypeStruct((R, W), jnp.float32),
    )(d2_rows, sel)


def kernel(stacked_coords, stacked_lattices, stacked_nbr_lists):
    B, N = stacked_coords.shape[0], stacked_coords.shape[1]
    M = stacked_nbr_lists.shape[2]

    nbr1 = stacked_nbr_lists[..., 0]
    nbr2 = stacked_nbr_lists[..., 1]
    coords_t = jnp.transpose(stacked_coords, (3, 0, 2, 1)).reshape(-1)
    nbr_t = jnp.stack([nbr1.reshape(B, N * M), nbr2.reshape(B, N * M)]).reshape(-1)
    lat_t = jnp.transpose(stacked_lattices, (2, 0, 1))
    lat_pad = jnp.broadcast_to(lat_t[..., None], (2, B, 3, 16)).reshape(-1)

    d2_1, d2_2 = _sc_dist2(coords_t, nbr_t, lat_pad, B, N, M)

    sel = np.zeros((M, M * NFILT), np.float32)
    for m in range(M):
        sel[m, m * NFILT:(m + 1) * NFILT] = 1.0
    sel = jnp.asarray(sel)

    bond_fea_1 = _tc_expand(d2_1.reshape(B * N, M), sel).reshape(B, N, M, NFILT)
    bond_fea_2 = _tc_expand(d2_2.reshape(B * N, M), sel).reshape(B, N, M, NFILT)
    return (nbr1, bond_fea_1, nbr2, bond_fea_2)


# speedup vs baseline: 25.1171x; 1.1925x over previous
import functools

import jax
import jax.numpy as jnp
import numpy as np
from jax import lax
from jax.experimental import pallas as pl
from jax.experimental.pallas import tpu as pltpu
from jax.experimental.pallas import tpu_sc as plsc

DMIN, DMAX, STEP = 0.0, 8.0, 0.25
VAR = STEP
NFILT = 33
NC, NS = 2, 16
SUBS_PER_CASE = 8


def _sc_dist2(coords_t, nbr_t, lat_pad, B, N, M):
    NM = N * M
    EPW = NM // SUBS_PER_CASE
    APW = N // SUBS_PER_CASE
    mesh = plsc.VectorSubcoreMesh(
        core_axis_name="c", subcore_axis_name="s", num_cores=NC, num_subcores=NS
    )

    @functools.partial(
        pl.kernel,
        mesh=mesh,
        compiler_params=pltpu.CompilerParams(needs_layout_passes=False),
        out_type=(
            jax.ShapeDtypeStruct((B * NM,), jnp.float32),
            jax.ShapeDtypeStruct((B * NM,), jnp.float32),
        ),
        scratch_types=[
            pltpu.VMEM((N,), jnp.float32),
            pltpu.VMEM((N,), jnp.float32),
            pltpu.VMEM((N,), jnp.float32),
            pltpu.VMEM((EPW,), jnp.int32),
            pltpu.VMEM((EPW,), jnp.float32),
            pltpu.VMEM((48,), jnp.float32),
        ],
    )
    def k(coords_hbm, nbr_hbm, lat_hbm, out1, out2, xv, yv, zv, idxv, ov, latv):
        c = lax.axis_index("c")
        s = lax.axis_index("s")
        b = s // SUBS_PER_CASE
        sub = s % SUBS_PER_CASE
        base = sub * EPW
        atoms0 = sub * APW
        case = c * B + b

        pltpu.sync_copy(coords_hbm.at[pl.ds((case * 3 + 0) * N, N)], xv)
        pltpu.sync_copy(coords_hbm.at[pl.ds((case * 3 + 1) * N, N)], yv)
        pltpu.sync_copy(coords_hbm.at[pl.ds((case * 3 + 2) * N, N)], zv)
        pltpu.sync_copy(nbr_hbm.at[pl.ds(case * NM + base, EPW)], idxv)
        pltpu.sync_copy(lat_hbm.at[pl.ds(case * 48, 48)], latv)

        lxv = latv[pl.ds(0, 16)]
        lyv = latv[pl.ds(16, 16)]
        lzv = latv[pl.ds(32, 16)]
        hxv = lxv * 0.5
        hyv = lyv * 0.5
        hzv = lzv * 0.5

        def body(a, _):
            gidx = jnp.full((16,), atoms0 + a, jnp.int32)
            ax = plsc.load_gather(xv, [gidx])
            ay = plsc.load_gather(yv, [gidx])
            az = plsc.load_gather(zv, [gidx])
            for h in range(M // 16):
                off = a * M + h * 16
                idx = idxv[pl.ds(off, 16)]
                dx = jnp.abs(ax - plsc.load_gather(xv, [idx]))
                dy = jnp.abs(ay - plsc.load_gather(yv, [idx]))
                dz = jnp.abs(az - plsc.load_gather(zv, [idx]))
                dx = jnp.where(dx > hxv, dx - lxv, dx)
                dy = jnp.where(dy > hyv, dy - lyv, dy)
                dz = jnp.where(dz > hzv, dz - lzv, dz)
                ov[pl.ds(off, 16)] = dx * dx + dy * dy + dz * dz
            return 0

        lax.fori_loop(0, APW, body, 0, unroll=2)

        @pl.when(c == 0)
        def _():
            pltpu.sync_copy(ov, out1.at[pl.ds(b * NM + base, EPW)])

        @pl.when(c == 1)
        def _():
            pltpu.sync_copy(ov, out2.at[pl.ds(b * NM + base, EPW)])

    return k(coords_t, nbr_t, lat_pad)


def _tc_expand(d2_flat):
    E = d2_flat.shape[0]
    EB = E // 128
    RQ = 40 if EB % 40 == 0 else 8
    assert EB % RQ == 0
    inv_var2 = 1.0 / (VAR * VAR)
    d2_rows = d2_flat.reshape(EB, 128)

    def body(d2_ref, o_ref):
        d = jnp.sqrt(d2_ref[...])
        dt = jnp.transpose(d)
        f = lax.broadcasted_iota(jnp.int32, (128, NFILT), 1).astype(jnp.float32) * STEP
        for q in range(RQ):
            dq = jnp.broadcast_to(dt[:, q:q + 1], (128, NFILT))
            diff = dq - f
            o_ref[q * 128:(q + 1) * 128, :] = jnp.exp(diff * diff * (-inv_var2))

    return pl.pallas_call(
        body,
        grid=(EB // RQ,),
        in_specs=[pl.BlockSpec((RQ, 128), lambda i: (i, 0))],
        out_specs=pl.BlockSpec((RQ * 128, NFILT), lambda i: (i, 0)),
        out_shape=jax.ShapeDtypeStruct((E, NFILT), jnp.float32),
    )(d2_rows)


def kernel(stacked_coords, stacked_lattices, stacked_nbr_lists):
    B, N = stacked_coords.shape[0], stacked_coords.shape[1]
    M = stacked_nbr_lists.shape[2]

    nbr1 = stacked_nbr_lists[..., 0]
    nbr2 = stacked_nbr_lists[..., 1]
    coords_t = jnp.transpose(stacked_coords, (3, 0, 2, 1)).reshape(-1)
    nbr_t = jnp.stack([nbr1.reshape(B, N * M), nbr2.reshape(B, N * M)]).reshape(-1)
    lat_t = jnp.transpose(stacked_lattices, (2, 0, 1))
    lat_pad = jnp.broadcast_to(lat_t[..., None], (2, B, 3, 16)).reshape(-1)

    d2_1, d2_2 = _sc_dist2(coords_t, nbr_t, lat_pad, B, N, M)

    bond_fea_1 = _tc_expand(d2_1).reshape(B, N, M, NFILT)
    bond_fea_2 = _tc_expand(d2_2).reshape(B, N, M, NFILT)
    return (nbr1, bond_fea_1, nbr2, bond_fea_2)
